# Initial kernel scaffold; baseline (speedup 1.0000x reference)
import jax
import jax.numpy as jnp
from jax.experimental import pallas as pl


def kernel(logits, temperature, top_p, frequency_penalties, presence_penalties, repetition_penalties, top_k, prompt_tokens, output_tokens, max_num_logprobs):
    raise NotImplementedError("write your pallas kernel here")


# speedup vs baseline: 5.3385x; 5.3385x over previous
import jax
import jax.numpy as jnp
from jax.experimental import pallas as pl

_EPS = 1e-5
_V = 100000
_VP = 100352
_R = 8
_NEG_BIG = -3.0e38


def _row_min_finite(x):
    return jnp.min(jnp.where(x > _NEG_BIG, x, jnp.inf), axis=-1, keepdims=True)


def _row_max(x):
    return jnp.max(x, axis=-1, keepdims=True)


def _rev_cumsum_excl(y):
    z = y
    s = 1
    r, vp = y.shape
    while s < vp:
        shifted = jnp.concatenate(
            [z[:, s:], jnp.zeros((r, s), z.dtype)], axis=-1)
        z = z + shifted
        s *= 2
    return z - y


def _sampler_kernel(logits_ref, q_ref, cnt_ref, rmask_ref,
                    temp_ref, topp_ref, fp_ref, pp_ref, rp_ref, k_ref,
                    samp_ref, tidx_ref, tlp_ref):
    x = logits_ref[...]
    temp = temp_ref[...]
    x = x / jnp.where(temp < _EPS, 1.0, temp)

    r, vp = x.shape
    iota = jax.lax.broadcasted_iota(jnp.int32, (r, vp), 1)
    neg_inf = jnp.float32(-jnp.inf)

    kk = k_ref[...]
    lo0 = _row_min_finite(x)
    hi0 = _row_max(x)
    hi0 = hi0 + jnp.abs(hi0) * 1e-6 + 1e-30

    def tk_body(_, carry):
        lo, hi = carry
        mid = 0.5 * (lo + hi)
        cnt = jnp.sum(jnp.where(x >= mid, 1.0, 0.0), axis=-1, keepdims=True)
        ok = cnt >= kk
        return (jnp.where(ok, mid, lo), jnp.where(ok, hi, mid))

    tk, _ = jax.lax.fori_loop(0, 52, tk_body, (lo0, hi0))
    x = jnp.where(x < tk, neg_inf, x)

    mx = _row_max(x)
    e = jnp.exp(x - mx)
    e = jnp.where(x > _NEG_BIG, e, 0.0)
    s = jnp.sum(e, axis=-1, keepdims=True)
    ps = topp_ref[...] * s

    plo0 = _row_min_finite(x) - 1.0
    phi0 = mx + jnp.abs(mx) * 1e-6 + 1e-30

    def tp_body(_, carry):
        lo, hi = carry
        mid = 0.5 * (lo + hi)
        mass = jnp.sum(jnp.where(x > mid, e, 0.0), axis=-1, keepdims=True)
        ok = mass < ps
        return (jnp.where(ok, lo, mid), jnp.where(ok, mid, hi))

    _, tstar = jax.lax.fori_loop(0, 52, tp_body, (plo0, phi0))

    mass_above = jnp.sum(jnp.where(x > tstar, e, 0.0), axis=-1, keepdims=True)
    p_t = jnp.exp(tstar - mx)
    eq = (x == tstar)
    eq_after = _rev_cumsum_excl(eq.astype(jnp.float32))
    keep_eq = eq & (mass_above + p_t * eq_after < ps)

    eqm = (x == mx)
    last_idx = jnp.max(jnp.where(eqm, iota, -1), axis=-1, keepdims=True)
    forced = eqm & (iota == last_idx)

    keep = (x > tstar) | keep_eq | forced
    x = jnp.where(keep, x, neg_inf)

    rp = rp_ref[...]
    finite = x > _NEG_BIG
    safe = jnp.where(finite, x, 0.0)
    pen = jnp.where(safe > 0, safe / rp, safe * rp)
    pen = jnp.where(finite, pen, x)
    cnts = cnt_ref[...]
    omask = cnts > 0
    x = jnp.where((rmask_ref[...] > 0) | omask, pen, x)
    x = x - fp_ref[...] * cnts
    x = x - pp_ref[...] * omask.astype(jnp.float32)

    mx2 = _row_max(x)
    e2 = jnp.exp(x - mx2)
    e2 = jnp.where(x > _NEG_BIG, e2, 0.0)
    s2 = jnp.sum(e2, axis=-1, keepdims=True)
    probs = e2 / s2

    pm = jnp.max(probs, axis=-1, keepdims=True)
    gidx = jnp.min(jnp.where(probs == pm, iota, vp), axis=-1, keepdims=True)
    qv = q_ref[...]
    ratio = probs / qv
    nan_mask = (probs == 0.0) & (qv == 0.0)
    ratio_clean = jnp.where(nan_mask, neg_inf, ratio)
    rm = jnp.max(ratio_clean, axis=-1, keepdims=True)
    ridx = jnp.min(jnp.where(ratio_clean == rm, iota, vp),
                   axis=-1, keepdims=True)
    first_nan = jnp.min(jnp.where(nan_mask, iota, vp), axis=-1, keepdims=True)
    ridx = jnp.where(first_nan < vp, first_nan, ridx)
    samp_ref[...] = jnp.where(temp < _EPS, gidx, ridx).astype(jnp.int32)

    lse = jnp.log(s2)
    lp = jnp.where(x > _NEG_BIG, (x - mx2) - lse, neg_inf)
    taken = jnp.zeros((r, vp), dtype=jnp.bool_)
    for j in range(20):
        cur = jnp.where(taken, neg_inf, lp)
        m = jnp.max(cur, axis=-1, keepdims=True)
        sel = (cur == m) & (~taken)
        idx = jnp.min(jnp.where(sel, iota, vp), axis=-1, keepdims=True)
        tidx_ref[:, j:j + 1] = idx.astype(jnp.int32)
        tlp_ref[:, j:j + 1] = m
        taken = taken | (iota == idx)


def kernel(logits, temperature, top_p, frequency_penalties, presence_penalties,
           repetition_penalties, top_k, prompt_tokens, output_tokens,
           max_num_logprobs):
    B, V = logits.shape
    pad = _VP - V

    neg_inf = jnp.float32(-jnp.inf)
    logits_p = jnp.pad(logits.astype(jnp.float32), ((0, 0), (0, pad)),
                       constant_values=-jnp.inf)

    q = jax.random.exponential(jax.random.key(42), (B, V), dtype=jnp.float32)
    q_p = jnp.pad(q, ((0, 0), (0, pad)), constant_values=1.0)

    def bin_counts(tokens):
        c = jnp.zeros((B, V + 1), dtype=jnp.int32)
        c = c.at[jnp.arange(B)[:, None], tokens].add(1)
        return c[:, :V]

    out_counts = bin_counts(output_tokens)
    prompt_counts = bin_counts(prompt_tokens)
    rep_mask = ((prompt_counts > 0) | (out_counts > 0)).astype(jnp.float32)
    cnt_p = jnp.pad(out_counts.astype(jnp.float32), ((0, 0), (0, pad)))
    rmask_p = jnp.pad(rep_mask, ((0, 0), (0, pad)))

    temp = temperature.reshape(B, 1)
    topp = top_p.reshape(B, 1)
    fp = frequency_penalties.reshape(B, 1)
    pp = presence_penalties.reshape(B, 1)
    rp = repetition_penalties.reshape(B, 1)
    kf = top_k.astype(jnp.float32).reshape(B, 1)

    row_spec = pl.BlockSpec((_R, _VP), lambda i: (i, 0))
    sc_spec = pl.BlockSpec((_R, 1), lambda i: (i, 0))

    samp, tidx, tlp = pl.pallas_call(
        _sampler_kernel,
        grid=(B // _R,),
        in_specs=[row_spec, row_spec, row_spec, row_spec,
                  sc_spec, sc_spec, sc_spec, sc_spec, sc_spec, sc_spec],
        out_specs=[sc_spec,
                   pl.BlockSpec((_R, 20), lambda i: (i, 0)),
                   pl.BlockSpec((_R, 20), lambda i: (i, 0))],
        out_shape=[jax.ShapeDtypeStruct((B, 1), jnp.int32),
                   jax.ShapeDtypeStruct((B, 20), jnp.int32),
                   jax.ShapeDtypeStruct((B, 20), jnp.float32)],
    )(logits_p, q_p, cnt_p, rmask_p, temp, topp, fp, pp, rp, kf)

    del neg_inf
    return samp.reshape(-1), tidx, tlp
